# Initial kernel scaffold; baseline (speedup 1.0000x reference)
import jax
import jax.numpy as jnp
from jax.experimental import pallas as pl


def kernel(input, adj, W, b):
    raise NotImplementedError("write your pallas kernel here")


# speedup vs baseline: 11.0310x; 11.0310x over previous
import functools
import jax
import jax.numpy as jnp
from jax import lax
from jax.experimental import pallas as pl
from jax.experimental.pallas import tpu as pltpu
from jax.experimental.pallas import tpu_sc as plsc

_N = 10000
_E = 320000
_D = 128
_G = 4
_OUT = 128
_CH = _D // _G
_NC = 2
_NS = 16
_EPS = _E // _NS
_B = 1000
_CNK = _EPS // _B
_T = 125
_NT = _B // _T
_ZR = 2 * _N // _NS


def _sc_aggregate(x_flat, rows_hbm, cols_hbm, zeros_hbm):
    mesh = plsc.VectorSubcoreMesh(core_axis_name="c", subcore_axis_name="s")

    @functools.partial(
        pl.kernel,
        out_type=jax.ShapeDtypeStruct((_N, _D), jnp.float32),
        mesh=mesh,
        scratch_types=[
            pltpu.VMEM((_NT, _T), jnp.int32),
            pltpu.VMEM((_NT, _T), jnp.int32),
            pltpu.VMEM((_B, _CH), jnp.float32),
            pltpu.VMEM_SHARED((2 * _N, _CH), jnp.float32),
            pltpu.SemaphoreType.DMA,
        ],
        compiler_params=pltpu.CompilerParams(use_tc_tiling_on_sc=False),
    )
    def k(x_hbm, r_hbm, c_hbm, z_hbm, agg_hbm, row_v, col_v, data_v, acc, sem):
        c = lax.axis_index("c")
        s = lax.axis_index("s")

        pltpu.sync_copy(z_hbm.at[pl.ds(s * _ZR, _ZR)], acc.at[pl.ds(s * _ZR, _ZR)])
        plsc.subcore_barrier()

        for gl in range(2):
            def chunk(kk, carry, gl=gl):
                cid = ((2 * c + gl) * _NS + s) * _CNK + kk
                pltpu.sync_copy(r_hbm.at[cid], row_v)
                pltpu.sync_copy(c_hbm.at[cid], col_v)
                for j in range(_NT):
                    pltpu.async_copy(
                        x_hbm.at[col_v.at[j]],
                        data_v.at[pl.ds(j * _T, _T)],
                        sem,
                    ).wait()
                    pltpu.sync_copy(
                        data_v.at[pl.ds(j * _T, _T)],
                        acc.at[row_v.at[j]],
                        add=True,
                    )
                return carry

            lax.fori_loop(0, _CNK, chunk, 0)

        plsc.subcore_barrier()

        g_out = 2 * c + s // 8
        node_base = (s % 8) * _ZR
        pltpu.sync_copy(
            acc.at[pl.ds(s * _ZR, _ZR)],
            agg_hbm.at[pl.ds(node_base, _ZR), pl.ds(g_out * _CH, _CH)],
        )

    return k(x_flat, rows_hbm, cols_hbm, zeros_hbm)


_BN = 1000


def _tc_matmul_body(agg_ref, w_ref, b_ref, out_ref):
    out_ref[...] = (
        jnp.dot(agg_ref[...], w_ref[...], preferred_element_type=jnp.float32)
        + b_ref[...]
    )


def _tc_matmul(agg, W, b):
    return pl.pallas_call(
        _tc_matmul_body,
        grid=(_N // _BN,),
        in_specs=[
            pl.BlockSpec((_BN, _D), lambda i: (i, 0)),
            pl.BlockSpec((_D, _OUT), lambda i: (0, 0)),
            pl.BlockSpec((1, _OUT), lambda i: (0, 0)),
        ],
        out_specs=pl.BlockSpec((_BN, _OUT), lambda i: (i, 0)),
        out_shape=jax.ShapeDtypeStruct((_N, _OUT), jnp.float32),
    )(agg, W, b.reshape(1, _OUT))


@jax.jit
def kernel(input, adj, W, b):
    x_flat = input.reshape(_N * _G, _CH)
    goff = jnp.arange(_G, dtype=jnp.int32)
    col_idx = adj[:, 1, :] * _G + goff[:, None]
    row_idx = adj[:, 0, :] + (goff[:, None] % 2) * _N
    rows_hbm = row_idx.reshape(_G * _NS * _CNK, _NT, _T)
    cols_hbm = col_idx.reshape(_G * _NS * _CNK, _NT, _T)
    zeros_hbm = jnp.zeros((2 * _N, _CH), jnp.float32)

    agg = _sc_aggregate(x_flat, rows_hbm, cols_hbm, zeros_hbm)
    return _tc_matmul(agg, W, b)


# speedup vs baseline: 22.6788x; 2.0559x over previous
import functools
import jax
import jax.numpy as jnp
from jax import lax
from jax.experimental import pallas as pl
from jax.experimental.pallas import tpu as pltpu
from jax.experimental.pallas import tpu_sc as plsc

_N = 10000
_E = 320000
_D = 128
_G = 4
_OUT = 128
_CH = _D // _G
_NC = 2
_NS = 16
_EPS = _E // _NS
_B = 1000
_CNK = _EPS // _B
_T = 125
_NT = _B // _T
_ZR = 2 * _N // _NS


def _sc_aggregate(x_flat, rows_hbm, cols_hbm, zeros_hbm):
    mesh = plsc.VectorSubcoreMesh(core_axis_name="c", subcore_axis_name="s")

    @functools.partial(
        pl.kernel,
        out_type=jax.ShapeDtypeStruct((_N, _D), jnp.float32),
        mesh=mesh,
        scratch_types=[
            pltpu.VMEM((2, _NT, _T), jnp.int32),
            pltpu.VMEM((2, _NT, _T), jnp.int32),
            pltpu.VMEM((_B, _CH), jnp.float32),
            pltpu.VMEM_SHARED((2 * _N, _CH), jnp.float32),
            pltpu.SemaphoreType.DMA,
            pltpu.SemaphoreType.DMA,
        ],
        compiler_params=pltpu.CompilerParams(use_tc_tiling_on_sc=False),
    )
    def k(x_hbm, r_hbm, c_hbm, z_hbm, agg_hbm, idx_r, idx_c, data_v, acc,
          sem_g, sem_i):
        c = lax.axis_index("c")
        s = lax.axis_index("s")

        pltpu.sync_copy(z_hbm.at[pl.ds(s * _ZR, _ZR)], acc.at[pl.ds(s * _ZR, _ZR)])
        plsc.subcore_barrier()

        TT = 2 * _CNK

        def cid_of(t):
            return ((2 * c + t // _CNK) * _NS + s) * _CNK + (t % _CNK)

        def fire_idx(t, slot):
            tw = t % TT
            pltpu.async_copy(r_hbm.at[cid_of(tw)], idx_r.at[slot], sem_i)
            pltpu.async_copy(c_hbm.at[cid_of(tw)], idx_c.at[slot], sem_i)

        def drain_idx(slot):
            pltpu.make_async_copy(r_hbm.at[0], idx_r.at[slot], sem_i).wait()
            pltpu.make_async_copy(c_hbm.at[0], idx_c.at[slot], sem_i).wait()

        fire_idx(0, 0)

        def chunk(t, carry):
            p = t % 2
            drain_idx(p)
            descs = [
                pltpu.async_copy(
                    x_hbm.at[idx_c.at[p, j]],
                    data_v.at[pl.ds(j * _T, _T)],
                    sem_g,
                )
                for j in range(_NT)
            ]
            fire_idx(t + 1, 1 - p)
            for j in range(_NT):
                descs[j].wait()
                pltpu.sync_copy(
                    data_v.at[pl.ds(j * _T, _T)],
                    acc.at[idx_r.at[p, j]],
                    add=True,
                )
            return carry

        lax.fori_loop(0, TT, chunk, 0)
        drain_idx(0)

        plsc.subcore_barrier()

        g_out = 2 * c + s // 8
        node_base = (s % 8) * _ZR
        pltpu.sync_copy(
            acc.at[pl.ds(s * _ZR, _ZR)],
            agg_hbm.at[pl.ds(node_base, _ZR), pl.ds(g_out * _CH, _CH)],
        )

    return k(x_flat, rows_hbm, cols_hbm, zeros_hbm)


_BN = 1000


def _tc_matmul_body(agg_ref, w_ref, b_ref, out_ref):
    out_ref[...] = (
        jnp.dot(agg_ref[...], w_ref[...], preferred_element_type=jnp.float32)
        + b_ref[...]
    )


def _tc_matmul(agg, W, b):
    return pl.pallas_call(
        _tc_matmul_body,
        grid=(_N // _BN,),
        in_specs=[
            pl.BlockSpec((_BN, _D), lambda i: (i, 0)),
            pl.BlockSpec((_D, _OUT), lambda i: (0, 0)),
            pl.BlockSpec((1, _OUT), lambda i: (0, 0)),
        ],
        out_specs=pl.BlockSpec((_BN, _OUT), lambda i: (i, 0)),
        out_shape=jax.ShapeDtypeStruct((_N, _OUT), jnp.float32),
    )(agg, W, b.reshape(1, _OUT))


@jax.jit
def kernel(input, adj, W, b):
    x_flat = input.reshape(_N * _G, _CH)
    goff = jnp.arange(_G, dtype=jnp.int32)
    col_idx = adj[:, 1, :] * _G + goff[:, None]
    row_idx = adj[:, 0, :] + (goff[:, None] % 2) * _N
    rows_hbm = row_idx.reshape(_G * _NS * _CNK, _NT, _T)
    cols_hbm = col_idx.reshape(_G * _NS * _CNK, _NT, _T)
    zeros_hbm = jnp.zeros((2 * _N, _CH), jnp.float32)

    agg = _sc_aggregate(x_flat, rows_hbm, cols_hbm, zeros_hbm)
    return _tc_matmul(agg, W, b)


# speedup vs baseline: 22.7288x; 1.0022x over previous
import functools
import jax
import jax.numpy as jnp
from jax import lax
from jax.experimental import pallas as pl
from jax.experimental.pallas import tpu as pltpu
from jax.experimental.pallas import tpu_sc as plsc

_N = 10000
_E = 320000
_D = 128
_G = 4
_OUT = 128
_CH = _D // _G
_NC = 2
_NS = 16
_EPS = _E // _NS
_B = 1000
_CNK = _EPS // _B
_T = 125
_NT = _B // _T
_ZR = 2 * _N // _NS


def _sc_aggregate(xg, adj6, zeros_hbm):
    mesh = plsc.VectorSubcoreMesh(core_axis_name="c", subcore_axis_name="s")

    @functools.partial(
        pl.kernel,
        out_type=jax.ShapeDtypeStruct((_N, _D), jnp.float32),
        mesh=mesh,
        scratch_types=[
            pltpu.VMEM((2, _NT, _T), jnp.int32),
            pltpu.VMEM((2, _NT, _T), jnp.int32),
            pltpu.VMEM((_B, _CH), jnp.float32),
            pltpu.VMEM_SHARED((2, _N, _CH), jnp.float32),
            pltpu.SemaphoreType.DMA,
            pltpu.SemaphoreType.DMA,
        ],
        compiler_params=pltpu.CompilerParams(use_tc_tiling_on_sc=False),
    )
    def k(xg_hbm, adj_hbm, z_hbm, agg_hbm, idx_r, idx_c, data_v, acc,
          sem_g, sem_i):
        c = lax.axis_index("c")
        s = lax.axis_index("s")

        gz = s // 8
        oz = (s % 8) * _ZR
        pltpu.sync_copy(z_hbm, acc.at[gz, pl.ds(oz, _ZR)])
        plsc.subcore_barrier()

        TT = 2 * _CNK

        def fire_idx(t, slot):
            tw = t % TT
            gl = tw // _CNK
            kk = tw % _CNK
            pltpu.async_copy(adj_hbm.at[2 * c + gl, 0, s, kk], idx_r.at[slot],
                             sem_i)
            pltpu.async_copy(adj_hbm.at[2 * c + gl, 1, s, kk], idx_c.at[slot],
                             sem_i)

        def drain_idx(slot):
            pltpu.make_async_copy(adj_hbm.at[0, 0, 0, 0], idx_r.at[slot],
                                  sem_i).wait()
            pltpu.make_async_copy(adj_hbm.at[0, 0, 0, 0], idx_c.at[slot],
                                  sem_i).wait()

        fire_idx(0, 0)

        def chunk(t, carry):
            p = t % 2
            gl = t // _CNK
            drain_idx(p)
            descs = [
                pltpu.async_copy(
                    xg_hbm.at[2 * c + gl].at[idx_c.at[p, j]],
                    data_v.at[pl.ds(j * _T, _T)],
                    sem_g,
                )
                for j in range(_NT)
            ]
            fire_idx(t + 1, 1 - p)
            for j in range(_NT):
                descs[j].wait()
                pltpu.sync_copy(
                    data_v.at[pl.ds(j * _T, _T)],
                    acc.at[gl].at[idx_r.at[p, j]],
                    add=True,
                )
            return carry

        lax.fori_loop(0, TT, chunk, 0)
        drain_idx(0)

        plsc.subcore_barrier()

        pltpu.sync_copy(
            acc.at[gz, pl.ds(oz, _ZR)],
            agg_hbm.at[pl.ds(oz, _ZR), pl.ds((2 * c + gz) * _CH, _CH)],
        )

    return k(xg, adj6, zeros_hbm)


_BN = 1000


def _tc_matmul_body(agg_ref, w_ref, b_ref, out_ref):
    out_ref[...] = (
        jnp.dot(agg_ref[...], w_ref[...], preferred_element_type=jnp.float32)
        + b_ref[...]
    )


def _tc_matmul(agg, W, b):
    return pl.pallas_call(
        _tc_matmul_body,
        grid=(_N // _BN,),
        in_specs=[
            pl.BlockSpec((_BN, _D), lambda i: (i, 0)),
            pl.BlockSpec((_D, _OUT), lambda i: (0, 0)),
            pl.BlockSpec((1, _OUT), lambda i: (0, 0)),
        ],
        out_specs=pl.BlockSpec((_BN, _OUT), lambda i: (i, 0)),
        out_shape=jax.ShapeDtypeStruct((_N, _OUT), jnp.float32),
    )(agg, W, b.reshape(1, _OUT))


@jax.jit
def kernel(input, adj, W, b):
    xg = input.reshape(_N, _G, _CH).transpose(1, 0, 2)
    adj6 = adj.reshape(_G, 2, _NS, _CNK, _NT, _T)
    zeros_hbm = jnp.zeros((_ZR, _CH), jnp.float32)

    agg = _sc_aggregate(xg, adj6, zeros_hbm)
    return _tc_matmul(agg, W, b)


# speedup vs baseline: 28.3474x; 1.2472x over previous
import functools
import jax
import jax.numpy as jnp
from jax import lax
from jax.experimental import pallas as pl
from jax.experimental.pallas import tpu as pltpu
from jax.experimental.pallas import tpu_sc as plsc

_N = 10000
_E = 320000
_D = 128
_G = 4
_OUT = 128
_CH = _D // _G
_NS = 16
_EPS = _E // _NS
_B = 1000
_CNK = _EPS // _B
_ZR = 2 * _N // _NS
_SPLITS = [(j * 128, 128) for j in range(7)] + [(896, 104)]
_NV = _B // 16


def _sc_aggregate(x2, adj, zeros_hbm):
    mesh = plsc.VectorSubcoreMesh(core_axis_name="c", subcore_axis_name="s")

    @functools.partial(
        pl.kernel,
        out_type=jax.ShapeDtypeStruct((_N, _D), jnp.float32),
        mesh=mesh,
        scratch_types=[
            pltpu.VMEM((2, _B), jnp.int32),
            pltpu.VMEM((2, _B), jnp.int32),
            pltpu.VMEM((_B,), jnp.int32),
            pltpu.VMEM((_B, _CH), jnp.float32),
            pltpu.VMEM_SHARED((2, _N, _CH), jnp.float32),
            pltpu.SemaphoreType.DMA,
            pltpu.SemaphoreType.DMA,
        ],
        compiler_params=pltpu.CompilerParams(use_tc_tiling_on_sc=False),
    )
    def k(x_hbm, adj_hbm, z_hbm, agg_hbm, idx_r, idx_c, idx_g, data_v, acc,
          sem_g, sem_i):
        c = lax.axis_index("c")
        s = lax.axis_index("s")

        gz = s // 8
        oz = (s % 8) * _ZR
        pltpu.sync_copy(z_hbm, acc.at[gz, pl.ds(oz, _ZR)])
        plsc.subcore_barrier()

        TT = 2 * _CNK

        def fire_idx(t, slot):
            tw = t % TT
            gl = tw // _CNK
            off = s * _EPS + (tw % _CNK) * _B
            pltpu.async_copy(adj_hbm.at[2 * c + gl, 0, pl.ds(off, _B)],
                             idx_r.at[slot], sem_i)
            pltpu.async_copy(adj_hbm.at[2 * c + gl, 1, pl.ds(off, _B)],
                             idx_c.at[slot], sem_i)

        def drain_idx(slot):
            pltpu.make_async_copy(adj_hbm.at[0, 0, pl.ds(0, _B)],
                                  idx_r.at[slot], sem_i).wait()
            pltpu.make_async_copy(adj_hbm.at[0, 0, pl.ds(0, _B)],
                                  idx_c.at[slot], sem_i).wait()

        fire_idx(0, 0)

        def chunk(t, carry):
            p = t % 2
            gl = t // _CNK
            g = 2 * c + gl
            drain_idx(p)
            for v in range(_NV):
                idx_g[pl.ds(16 * v, 16)] = idx_c[p, pl.ds(16 * v, 16)] * _G + g
            idx_g[pl.ds(_B - 16, 16)] = idx_c[p, pl.ds(_B - 16, 16)] * _G + g
            descs = [
                pltpu.async_copy(
                    x_hbm.at[idx_g.at[pl.ds(o, n)]],
                    data_v.at[pl.ds(o, n)],
                    sem_g,
                )
                for o, n in _SPLITS
            ]
            fire_idx(t + 1, 1 - p)
            for d, (o, n) in zip(descs, _SPLITS):
                d.wait()
                pltpu.sync_copy(
                    data_v.at[pl.ds(o, n)],
                    acc.at[gl].at[idx_r.at[p, pl.ds(o, n)]],
                    add=True,
                )
            return carry

        lax.fori_loop(0, TT, chunk, 0)
        drain_idx(0)

        plsc.subcore_barrier()

        pltpu.sync_copy(
            acc.at[gz, pl.ds(oz, _ZR)],
            agg_hbm.at[pl.ds(oz, _ZR), pl.ds((2 * c + gz) * _CH, _CH)],
        )

    return k(x2, adj, zeros_hbm)


_BN = 1000


def _tc_matmul_body(agg_ref, w_ref, b_ref, out_ref):
    out_ref[...] = (
        jnp.dot(agg_ref[...], w_ref[...], preferred_element_type=jnp.float32)
        + b_ref[...]
    )


def _tc_matmul(agg, W, b):
    return pl.pallas_call(
        _tc_matmul_body,
        grid=(_N // _BN,),
        in_specs=[
            pl.BlockSpec((_BN, _D), lambda i: (i, 0)),
            pl.BlockSpec((_D, _OUT), lambda i: (0, 0)),
            pl.BlockSpec((1, _OUT), lambda i: (0, 0)),
        ],
        out_specs=pl.BlockSpec((_BN, _OUT), lambda i: (i, 0)),
        out_shape=jax.ShapeDtypeStruct((_N, _OUT), jnp.float32),
    )(agg, W, b.reshape(1, _OUT))


@jax.jit
def kernel(input, adj, W, b):
    x2 = input.reshape(_N * _G, _CH)
    zeros_hbm = jnp.zeros((_ZR, _CH), jnp.float32)
    agg = _sc_aggregate(x2, adj, zeros_hbm)
    return _tc_matmul(agg, W, b)


# speedup vs baseline: 29.7095x; 1.0481x over previous
import functools
import jax
import jax.numpy as jnp
from jax import lax
from jax.experimental import pallas as pl
from jax.experimental.pallas import tpu as pltpu
from jax.experimental.pallas import tpu_sc as plsc

_N = 10000
_E = 320000
_D = 128
_G = 4
_OUT = 128
_CH = _D // _G
_NS = 16
_EPS = _E // _NS
_B = 1000
_CNK = _EPS // _B
_ZR = 2 * _N // _NS
_SPLITS = [(j * 128, 128) for j in range(7)] + [(896, 104)]
_NV = _B // 16


def _sc_aggregate(x2, adj, zeros_hbm):
    mesh = plsc.VectorSubcoreMesh(core_axis_name="c", subcore_axis_name="s")

    @functools.partial(
        pl.kernel,
        out_type=jax.ShapeDtypeStruct((_N, _D), jnp.float32),
        mesh=mesh,
        scratch_types=[
            pltpu.VMEM((2, _B), jnp.int32),
            pltpu.VMEM((2, _B), jnp.int32),
            pltpu.VMEM((2, _B), jnp.int32),
            pltpu.VMEM((_B, _CH), jnp.float32),
            pltpu.VMEM_SHARED((2, _N, _CH), jnp.float32),
            pltpu.SemaphoreType.DMA,
            pltpu.SemaphoreType.DMA,
        ],
        compiler_params=pltpu.CompilerParams(use_tc_tiling_on_sc=False),
    )
    def k(x_hbm, adj_hbm, z_hbm, agg_hbm, idx_r, idx_c, idx_g, data_v, acc,
          sem_g, sem_i):
        c = lax.axis_index("c")
        s = lax.axis_index("s")

        gz = s // 8
        oz = (s % 8) * _ZR
        pltpu.sync_copy(z_hbm, acc.at[gz, pl.ds(oz, _ZR)])
        plsc.subcore_barrier()

        TT = 2 * _CNK

        def fire_idx(t, slot):
            tw = t % TT
            gl = tw // _CNK
            off = s * _EPS + (tw % _CNK) * _B
            pltpu.async_copy(adj_hbm.at[2 * (2 * c + gl), pl.ds(off, _B)],
                             idx_r.at[slot], sem_i)
            pltpu.async_copy(adj_hbm.at[2 * (2 * c + gl) + 1, pl.ds(off, _B)],
                             idx_c.at[slot], sem_i)

        def drain_idx(slot):
            pltpu.make_async_copy(adj_hbm.at[0, pl.ds(0, _B)],
                                  idx_r.at[slot], sem_i).wait()
            pltpu.make_async_copy(adj_hbm.at[0, pl.ds(0, _B)],
                                  idx_c.at[slot], sem_i).wait()

        def transform(t, slot):
            g = 2 * c + (t % TT) // _CNK
            for v in range(_NV):
                idx_g[slot, pl.ds(16 * v, 16)] = (
                    idx_c[slot, pl.ds(16 * v, 16)] * _G + g)
            idx_g[slot, pl.ds(_B - 16, 16)] = (
                idx_c[slot, pl.ds(_B - 16, 16)] * _G + g)

        fire_idx(0, 0)
        drain_idx(0)
        transform(0, 0)
        fire_idx(1, 1)

        def chunk(t, carry):
            p = t % 2
            gl = t // _CNK
            descs = [
                pltpu.async_copy(
                    x_hbm.at[idx_g.at[p, pl.ds(o, n)]],
                    data_v.at[pl.ds(o, n)],
                    sem_g,
                )
                for o, n in _SPLITS
            ]
            drain_idx(1 - p)
            transform(t + 1, 1 - p)
            for d, (o, n) in zip(descs, _SPLITS):
                d.wait()
                pltpu.sync_copy(
                    data_v.at[pl.ds(o, n)],
                    acc.at[gl].at[idx_r.at[p, pl.ds(o, n)]],
                    add=True,
                )
            fire_idx(t + 2, p)
            return carry

        lax.fori_loop(0, TT, chunk, 0)
        drain_idx(1)

        plsc.subcore_barrier()

        pltpu.sync_copy(
            acc.at[gz, pl.ds(oz, _ZR)],
            agg_hbm.at[pl.ds(oz, _ZR), pl.ds((2 * c + gz) * _CH, _CH)],
        )

    return k(x2, adj, zeros_hbm)


_BN = 1000


def _tc_matmul_body(agg_ref, w_ref, b_ref, out_ref):
    out_ref[...] = (
        jnp.dot(agg_ref[...], w_ref[...], preferred_element_type=jnp.float32)
        + b_ref[...]
    )


def _tc_matmul(agg, W, b):
    return pl.pallas_call(
        _tc_matmul_body,
        grid=(_N // _BN,),
        in_specs=[
            pl.BlockSpec((_BN, _D), lambda i: (i, 0)),
            pl.BlockSpec((_D, _OUT), lambda i: (0, 0)),
            pl.BlockSpec((1, _OUT), lambda i: (0, 0)),
        ],
        out_specs=pl.BlockSpec((_BN, _OUT), lambda i: (i, 0)),
        out_shape=jax.ShapeDtypeStruct((_N, _OUT), jnp.float32),
    )(agg, W, b.reshape(1, _OUT))


@jax.jit
def kernel(input, adj, W, b):
    x2 = input.reshape(_N * _G, _CH)
    adj8 = adj.reshape(2 * _G, _E)
    zeros_hbm = jnp.zeros((_ZR, _CH), jnp.float32)
    agg = _sc_aggregate(x2, adj8, zeros_hbm)
    return _tc_matmul(agg, W, b)
